# baseline (speedup 1.0000x reference)
import functools
import jax
import jax.numpy as jnp
from jax import lax
from jax.experimental import pallas as pl
from jax.experimental.pallas import tpu as pltpu
from jax.experimental.pallas import tpu_sc as plsc

B = 16384
G = 16
K = 64
F = 1024
NC, NS = 2, 16
NW = NC * NS
SC_ROWS = 6144
TC_ROWS = B - SC_ROWS
RW = SC_ROWS // NW
CH = 32
NCHUNK = RW // CH
RPI = 8
BB = 1024


def _sc_body(w_hbm, x_hbm, out_hbm, w_v, tt_v, st_v, kidx_v, buf0, buf1, ob,
             sem0, sem1):
    cid = lax.axis_index("c")
    sid = lax.axis_index("s")
    wid = cid * NS + sid
    base = wid * RW

    lanes = lax.iota(jnp.int32, 16)

    pltpu.sync_copy(w_hbm, w_v)

    def p1(j, mc):
        v = plsc.load_gather(w_v, [lanes, jnp.full((16,), j, jnp.int32)])
        tt_v[pl.ds(16 * j, 16)] = v
        return jnp.maximum(mc, v)
    m = lax.fori_loop(0, K, p1, jnp.full((16,), -jnp.inf, jnp.float32))

    def p2(j, sc):
        ev = jnp.exp(tt_v[pl.ds(16 * j, 16)] - m)
        sidx = ((j - lanes) & (K - 1)) * 16 + lanes
        plsc.store_scatter(st_v, [sidx], ev)
        return sc + ev
    s = lax.fori_loop(0, K, p2, jnp.zeros((16,), jnp.float32))
    inv = 1.0 / s

    def p3(j, t):
        st_v[pl.ds(16 * j, 16)] = st_v[pl.ds(16 * j, 16)] * inv
        kidx_v[pl.ds(16 * j, 16)] = lanes * K + ((j + lanes) & (K - 1))
        return t
    lax.fori_loop(0, K, p3, 0)

    def start(c, buf, sem):
        pltpu.make_async_copy(x_hbm.at[pl.ds(base + c * CH, CH)], buf, sem).start()

    def wait(c, buf, sem):
        pltpu.make_async_copy(x_hbm.at[pl.ds(base + c * CH, CH)], buf, sem).wait()

    def compute(buf, c):
        def row_body(i, _):
            r0 = i * RPI
            rfs = [jnp.full((16,), r0 + t, jnp.int32) for t in range(RPI)]
            acc = [jnp.zeros((16,), jnp.float32) for _ in range(RPI)]
            for j in range(K):
                s = st_v[pl.ds(16 * j, 16)]
                cj = kidx_v[pl.ds(16 * j, 16)]
                for t in range(RPI):
                    acc[t] = acc[t] + plsc.load_gather(buf, [rfs[t], cj]) * s
            for t in range(RPI):
                ob[r0 + t, :] = acc[t]
            return 0
        lax.fori_loop(0, CH // RPI, row_body, 0)
        pltpu.sync_copy(ob, out_hbm.at[pl.ds(base + c * CH, CH)])

    start(0, buf0, sem0)

    def outer(p, _):
        c0 = p * 2
        start(c0 + 1, buf1, sem1)
        wait(c0, buf0, sem0)
        compute(buf0, c0)

        @pl.when(p + 1 < NCHUNK // 2)
        def _():
            start(c0 + 2, buf0, sem0)

        wait(c0 + 1, buf1, sem1)
        compute(buf1, c0 + 1)
        return 0

    lax.fori_loop(0, NCHUNK // 2, outer, 0)


def _sc_call(w, x):
    mesh = plsc.VectorSubcoreMesh(core_axis_name="c", subcore_axis_name="s",
                                  num_cores=NC, num_subcores=NS)
    k = functools.partial(
        pl.kernel,
        out_type=jax.ShapeDtypeStruct((SC_ROWS, G), jnp.float32),
        mesh=mesh,
        scratch_types=[
            pltpu.VMEM((G, K), jnp.float32),
            pltpu.VMEM((K * 16,), jnp.float32),
            pltpu.VMEM((K * 16,), jnp.float32),
            pltpu.VMEM((K * 16,), jnp.int32),
            pltpu.VMEM((CH, F), jnp.float32),
            pltpu.VMEM((CH, F), jnp.float32),
            pltpu.VMEM((CH, G), jnp.float32),
            pltpu.SemaphoreType.DMA,
            pltpu.SemaphoreType.DMA,
        ],
        compiler_params=pltpu.CompilerParams(needs_layout_passes=False),
    )(_sc_body)
    return k(w, x)


def _tc_body(w_ref, x_ref, o_ref):
    w = w_ref[:]
    row_grp = lax.broadcasted_iota(jnp.int32, (F, G), 0) // K
    col = lax.broadcasted_iota(jnp.int32, (F, G), 1)
    mask = row_grp == col
    wb = jnp.where(mask, w, -jnp.inf)
    gm = jnp.max(wb, axis=0, keepdims=True)
    e = jnp.exp(wb - gm)
    Wm = e / jnp.sum(e, axis=0, keepdims=True)
    o_ref[:] = jnp.dot(x_ref[:], Wm, preferred_element_type=jnp.float32)


def _tc_call(wcol, x):
    off = SC_ROWS // BB
    return pl.pallas_call(
        _tc_body,
        grid=(TC_ROWS // BB,),
        in_specs=[
            pl.BlockSpec((F, 1), lambda i: (0, 0)),
            pl.BlockSpec((BB, F), lambda i: (i + off, 0)),
        ],
        out_specs=pl.BlockSpec((BB, G), lambda i: (i + off, 0)),
        out_shape=jax.ShapeDtypeStruct((B, G), jnp.float32),
    )(wcol, x)


@jax.jit
def _run(w, x):
    sc_out = _sc_call(w, x)
    tc_out = _tc_call(w.reshape(F, 1), x)
    return lax.dynamic_update_slice(tc_out, sc_out, (0, 0))


def kernel(gene_set_features, attention_weights):
    return _run(attention_weights, gene_set_features)


# speedup vs baseline: 1.1060x; 1.1060x over previous
import functools
import jax
import jax.numpy as jnp
from jax import lax
from jax.experimental import pallas as pl
from jax.experimental.pallas import tpu as pltpu
from jax.experimental.pallas import tpu_sc as plsc

B = 16384
G = 16
K = 64
F = 1024
NC, NS = 2, 16
NW = NC * NS
SC_ROWS = 4096
TC_ROWS = B - SC_ROWS
RW = SC_ROWS // NW
CH = 32
NCHUNK = RW // CH
RPI = 8
BB = 1024


def _sc_body(w_hbm, x_hbm, out_hbm, w_v, tt_v, st_v, kidx_v, buf0, buf1, ob,
             sem0, sem1):
    cid = lax.axis_index("c")
    sid = lax.axis_index("s")
    wid = cid * NS + sid
    base = wid * RW

    lanes = lax.iota(jnp.int32, 16)

    pltpu.sync_copy(w_hbm, w_v)

    def p1(j, mc):
        v = plsc.load_gather(w_v, [lanes, jnp.full((16,), j, jnp.int32)])
        tt_v[pl.ds(16 * j, 16)] = v
        return jnp.maximum(mc, v)
    m = lax.fori_loop(0, K, p1, jnp.full((16,), -jnp.inf, jnp.float32))

    def p2(j, sc):
        ev = jnp.exp(tt_v[pl.ds(16 * j, 16)] - m)
        sidx = ((j - lanes) & (K - 1)) * 16 + lanes
        plsc.store_scatter(st_v, [sidx], ev)
        return sc + ev
    s = lax.fori_loop(0, K, p2, jnp.zeros((16,), jnp.float32))
    inv = 1.0 / s

    def p3(j, t):
        st_v[pl.ds(16 * j, 16)] = st_v[pl.ds(16 * j, 16)] * inv
        kidx_v[pl.ds(16 * j, 16)] = lanes * K + ((j + lanes) & (K - 1))
        return t
    lax.fori_loop(0, K, p3, 0)

    def start(c, buf, sem):
        pltpu.make_async_copy(x_hbm.at[pl.ds(base + c * CH, CH)], buf, sem).start()

    def wait(c, buf, sem):
        pltpu.make_async_copy(x_hbm.at[pl.ds(base + c * CH, CH)], buf, sem).wait()

    def compute(buf, c):
        def row_body(i, _):
            r0 = i * RPI
            rfs = [jnp.full((16,), r0 + t, jnp.int32) for t in range(RPI)]
            acc = [jnp.zeros((16,), jnp.float32) for _ in range(RPI)]
            for j in range(K):
                s = st_v[pl.ds(16 * j, 16)]
                cj = kidx_v[pl.ds(16 * j, 16)]
                for t in range(RPI):
                    acc[t] = acc[t] + plsc.load_gather(buf, [rfs[t], cj]) * s
            for t in range(RPI):
                ob[r0 + t, :] = acc[t]
            return 0
        lax.fori_loop(0, CH // RPI, row_body, 0)
        pltpu.sync_copy(ob, out_hbm.at[pl.ds(base + c * CH, CH)])

    start(0, buf0, sem0)

    def outer(p, _):
        c0 = p * 2
        start(c0 + 1, buf1, sem1)
        wait(c0, buf0, sem0)
        compute(buf0, c0)

        @pl.when(p + 1 < NCHUNK // 2)
        def _():
            start(c0 + 2, buf0, sem0)

        wait(c0 + 1, buf1, sem1)
        compute(buf1, c0 + 1)
        return 0

    lax.fori_loop(0, NCHUNK // 2, outer, 0)


def _sc_call(w, x):
    mesh = plsc.VectorSubcoreMesh(core_axis_name="c", subcore_axis_name="s",
                                  num_cores=NC, num_subcores=NS)
    k = functools.partial(
        pl.kernel,
        out_type=jax.ShapeDtypeStruct((SC_ROWS, G), jnp.float32),
        mesh=mesh,
        scratch_types=[
            pltpu.VMEM((G, K), jnp.float32),
            pltpu.VMEM((K * 16,), jnp.float32),
            pltpu.VMEM((K * 16,), jnp.float32),
            pltpu.VMEM((K * 16,), jnp.int32),
            pltpu.VMEM((CH, F), jnp.float32),
            pltpu.VMEM((CH, F), jnp.float32),
            pltpu.VMEM((CH, G), jnp.float32),
            pltpu.SemaphoreType.DMA,
            pltpu.SemaphoreType.DMA,
        ],
        compiler_params=pltpu.CompilerParams(needs_layout_passes=False),
    )(_sc_body)
    return k(w, x)


def _tc_body(w_ref, x_ref, o_ref):
    w = w_ref[:]
    row_grp = lax.broadcasted_iota(jnp.int32, (F, G), 0) // K
    col = lax.broadcasted_iota(jnp.int32, (F, G), 1)
    mask = row_grp == col
    wb = jnp.where(mask, w, -jnp.inf)
    gm = jnp.max(wb, axis=0, keepdims=True)
    e = jnp.exp(wb - gm)
    Wm = e / jnp.sum(e, axis=0, keepdims=True)
    o_ref[:] = jnp.dot(x_ref[:], Wm, preferred_element_type=jnp.float32)


def _tc_call(wcol, x):
    off = SC_ROWS // BB
    return pl.pallas_call(
        _tc_body,
        grid=(TC_ROWS // BB,),
        in_specs=[
            pl.BlockSpec((F, 1), lambda i: (0, 0)),
            pl.BlockSpec((BB, F), lambda i: (i + off, 0)),
        ],
        out_specs=pl.BlockSpec((BB, G), lambda i: (i + off, 0)),
        out_shape=jax.ShapeDtypeStruct((B, G), jnp.float32),
    )(wcol, x)


@jax.jit
def _run(w, x):
    sc_out = _sc_call(w, x)
    tc_out = _tc_call(w.reshape(F, 1), x)
    return lax.dynamic_update_slice(tc_out, sc_out, (0, 0))


def kernel(gene_set_features, attention_weights):
    return _run(attention_weights, gene_set_features)
